# Initial kernel scaffold; baseline (speedup 1.0000x reference)
import jax
import jax.numpy as jnp
from jax.experimental import pallas as pl


def kernel(x):
    raise NotImplementedError("write your pallas kernel here")


# speedup vs baseline: 18.9730x; 18.9730x over previous
import functools
import jax
import jax.numpy as jnp
import numpy as np
from jax import lax
from jax.experimental import pallas as pl
from jax.experimental.pallas import tpu as pltpu
from jax.experimental.pallas import tpu_sc as plsc

N_ROWS = 96
ROW_LEN = 512 * 512
K_LO = 5243
K_HI = 256900

NC = 2
NS = 16
NW = NC * NS
ROWS_PER_W = N_ROWS // NW

CHUNK = 8192
NCH_A = (ROW_LEN // 2) // CHUNK
NCH_B = ROW_LEN // CHUNK
NBUF = 2
UNROLL = 4

NB0 = 16384
CAP = 32768

_MIN_I32 = np.int32(-2147483648)


def _zero(hist, nwords):
    z = jnp.zeros((16,), jnp.int32)

    def body(i, carry):
        for u in range(UNROLL):
            hist[pl.ds((i * UNROLL + u) * 16, 16)] = z
        return carry

    lax.fori_loop(0, nwords // (16 * UNROLL), body, 0)


def _find2(hist, nbins, k_a, k_b):
    iota = lax.iota(jnp.int32, 16)

    def body(i, c):
        tot, bin_a, r_a, c_a, bin_b, r_b, c_b = c
        v = hist[pl.ds(i * 16, 16)]
        cs = plsc.cumsum(v)
        vt = jnp.sum(v)
        excl = (tot + cs) - v
        newtot = tot + vt

        def pick(k, b_, r_, n_):
            hit = jnp.logical_and(tot <= k, k < newtot)
            sel = jnp.logical_and(excl <= k, k < excl + v)
            lane = jnp.sum(jnp.where(sel, iota, 0))
            rr = jnp.sum(jnp.where(sel, k - excl, 0))
            cc = jnp.sum(jnp.where(sel, v, 0))
            return (jnp.where(hit, i * 16 + lane, b_),
                    jnp.where(hit, rr, r_),
                    jnp.where(hit, cc, n_))

        bin_a, r_a, c_a = pick(k_a, bin_a, r_a, c_a)
        bin_b, r_b, c_b = pick(k_b, bin_b, r_b, c_b)
        return (newtot, bin_a, r_a, c_a, bin_b, r_b, c_b)

    z = jnp.int32(0)
    c = lax.fori_loop(0, nbins // 16, body, (z,) * 7)
    return c[1], c[2], c[3], c[4], c[5], c[6]


def _find1(hist, base, nbins, k):
    iota = lax.iota(jnp.int32, 16)

    def body(i, c):
        tot, bin_, r_ = c
        v = hist[pl.ds(base + i * 16, 16)]
        cs = plsc.cumsum(v)
        vt = jnp.sum(v)
        excl = (tot + cs) - v
        newtot = tot + vt
        hit = jnp.logical_and(tot <= k, k < newtot)
        sel = jnp.logical_and(excl <= k, k < excl + v)
        lane = jnp.sum(jnp.where(sel, iota, 0))
        rr = jnp.sum(jnp.where(sel, k - excl, 0))
        return (newtot,
                jnp.where(hit, i * 16 + lane, bin_),
                jnp.where(hit, rr, r_))

    z = jnp.int32(0)
    c = lax.fori_loop(0, nbins // 16, body, (z, z, z))
    return c[1], c[2]


def _sel_body(hpk_hbm, key_hbm, out_hbm, bufs, h_big, cb_lo, cb_hi, orow,
              sem0, sem1):
    wid = lax.axis_index("s") * NC + lax.axis_index("c")
    iota = lax.iota(jnp.int32, 16)
    ones = jnp.ones((16,), jnp.int32)
    sems = (sem0, sem1)

    def stream(src, base, n_chunks, carry0, update):
        def dma_start(ci, b):
            pltpu.make_async_copy(
                src.at[pl.ds(base + ci * CHUNK, CHUNK)],
                bufs.at[b], sems[b]).start()

        def dma_wait(b):
            pltpu.make_async_copy(
                src.at[pl.ds(base, CHUNK)], bufs.at[b], sems[b]).wait()

        def process(b, carry):
            def inner(i, c):
                for u in range(UNROLL):
                    v = bufs[b, pl.ds((i * UNROLL + u) * 16, 16)]
                    c = update(v, c, u)
                return c

            return lax.fori_loop(0, CHUNK // (16 * UNROLL), inner, carry)

        for b in range(NBUF - 1):
            dma_start(b, b)

        def g_body(g, carry):
            for b in range(NBUF):
                ci = g * NBUF + b
                dma_start(jnp.minimum(ci + NBUF - 1, n_chunks - 1),
                          (b + NBUF - 1) % NBUF)
                dma_wait(b)
                carry = process(b, carry)
            return carry

        carry = lax.fori_loop(0, n_chunks // NBUF, g_body, carry0)
        for b in range(NBUF - 1):
            dma_wait(b)
        return carry

    def do_row(j, carry):
        row = wid * ROWS_PER_W + j
        base_a = row * (ROW_LEN // 2)
        base_b = row * ROW_LEN

        _zero(h_big, NB0)

        def upd_a(v, c, u):
            d_lo = jnp.bitwise_and(v, jnp.int32(0xFFFF))
            d_hi = lax.shift_right_logical(v, 16)
            plsc.addupdate_scatter(h_big, [d_lo], ones)
            plsc.addupdate_scatter(h_big, [d_hi], ones)
            return c

        stream(hpk_hbm, base_a, NCH_A, 0, upd_a)

        b0_lo, r_lo, c_lo, b0_hi, r_hi, c_hi = _find2(
            h_big, NB0, jnp.int32(K_LO), jnp.int32(K_HI))

        def upd_b(v, c, u):
            ofs_l, ofs_h = c
            hi = lax.shift_right_logical(v, 18)
            val = jnp.bitwise_and(v, jnp.int32(0x3FFFF))
            ml = hi == b0_lo
            mh = hi == b0_hi
            plsc.store_compressed(cb_lo.at[pl.ds(ofs_l, 16)], val, mask=ml)
            plsc.store_compressed(cb_hi.at[pl.ds(ofs_h, 16)], val, mask=mh)
            pl_ = plsc.all_reduce_population_count(ml)[0]
            ph_ = plsc.all_reduce_population_count(mh)[0]
            return (jnp.minimum(ofs_l + pl_, jnp.int32(CAP)),
                    jnp.minimum(ofs_h + ph_, jnp.int32(CAP)))

        stream(key_hbm, base_b, NCH_B, (jnp.int32(0), jnp.int32(0)), upd_b)

        def mini(cbuf, m, r):
            _zero(h_big, 1024)
            nv = lax.div(m + 15, jnp.int32(16))

            def p1(i, c):
                v = cbuf[pl.ds(i * 16, 16)]
                lm = (iota + i * 16) < m
                d = lax.shift_right_logical(v, 9)
                plsc.addupdate_scatter(h_big, [d], ones, mask=lm)
                return c

            lax.fori_loop(0, nv, p1, 0)
            b1, r2 = _find1(h_big, 0, 512, r)

            def p2(i, c):
                v = cbuf[pl.ds(i * 16, 16)]
                lm = (iota + i * 16) < m
                mid = lax.shift_right_logical(v, 9)
                d2 = jnp.bitwise_and(v, jnp.int32(0x1FF)) + jnp.int32(512)
                plsc.addupdate_scatter(
                    h_big, [d2], ones,
                    mask=jnp.logical_and(lm, mid == b1))
                return c

            lax.fori_loop(0, nv, p2, 0)
            b2, _ = _find1(h_big, 512, 512, r2)
            return jnp.bitwise_or(lax.shift_left(b1, 9), b2)

        def fallback(b0, r):
            _zero(h_big, 1024)

            def f1(v, c, u):
                hi = lax.shift_right_logical(v, 18)
                d = jnp.bitwise_and(lax.shift_right_logical(v, 9),
                                    jnp.int32(0x1FF))
                plsc.addupdate_scatter(h_big, [d], ones, mask=(hi == b0))
                return c

            stream(key_hbm, base_b, NCH_B, 0, f1)
            b1, r2 = _find1(h_big, 0, 512, r)

            def f2(v, c, u):
                hi = lax.shift_right_logical(v, 18)
                mid = jnp.bitwise_and(lax.shift_right_logical(v, 9),
                                      jnp.int32(0x1FF))
                d2 = jnp.bitwise_and(v, jnp.int32(0x1FF)) + jnp.int32(512)
                plsc.addupdate_scatter(
                    h_big, [d2], ones,
                    mask=jnp.logical_and(hi == b0, mid == b1))
                return c

            stream(key_hbm, base_b, NCH_B, 0, f2)
            b2, _ = _find1(h_big, 512, 512, r2)
            return jnp.bitwise_or(lax.shift_left(b1, 9), b2)

        lo18_lo = lax.cond(c_lo <= CAP,
                           lambda _: mini(cb_lo, c_lo, r_lo),
                           lambda _: fallback(b0_lo, r_lo), 0)
        lo18_hi = lax.cond(c_hi <= CAP,
                           lambda _: mini(cb_hi, c_hi, r_hi),
                           lambda _: fallback(b0_hi, r_hi), 0)

        k_lo = jnp.bitwise_or(lax.shift_left(b0_lo, 18), lo18_lo)
        k_hi = jnp.bitwise_or(lax.shift_left(b0_hi, 18), lo18_hi)

        kvec = jnp.where(iota == 0, k_lo, jnp.where(iota == 1, k_hi, 0))
        bvec = jnp.where(kvec < 0, kvec ^ _MIN_I32, ~kvec)
        fvec = plsc.bitcast(bvec, jnp.float32)
        orow[...] = fvec
        pltpu.sync_copy(orow, out_hbm.at[pl.ds(row * 16, 16)])
        return carry

    lax.fori_loop(0, ROWS_PER_W, do_row, 0)


_select = functools.partial(
    pl.kernel,
    out_type=jax.ShapeDtypeStruct((N_ROWS * 16,), jnp.float32),
    mesh=plsc.VectorSubcoreMesh(core_axis_name="c", subcore_axis_name="s"),
    scratch_types=[
        pltpu.VMEM((NBUF, CHUNK), jnp.int32),
        pltpu.VMEM((NB0,), jnp.int32),
        pltpu.VMEM((CAP + 16,), jnp.int32),
        pltpu.VMEM((CAP + 16,), jnp.int32),
        pltpu.VMEM((16,), jnp.float32),
        pltpu.SemaphoreType.DMA,
        pltpu.SemaphoreType.DMA,
    ],
    compiler_params=pltpu.CompilerParams(needs_layout_passes=False),
)(_sel_body)


def _prep_body(x_ref, key_ref, hpk_ref, mm_ref):
    xb = x_ref[...]
    b = lax.bitcast_convert_type(xb, jnp.int32)
    key = b ^ (lax.shift_right_arithmetic(b, 31) | _MIN_I32)
    key_ref[...] = key
    hi_a = lax.shift_right_logical(key[:, :256, :], 18)
    hi_b = lax.shift_right_logical(key[:, 256:, :], 18)
    hpk_ref[...] = jnp.bitwise_or(hi_a, lax.shift_left(hi_b, 16))
    l = lax.broadcasted_iota(jnp.int32, (1, 1, 128), 2)
    mm_ref[...] = jnp.where(l == 0, jnp.min(xb),
                            jnp.where(l == 1, jnp.max(xb), jnp.float32(0.0)))


def _norm_body(sc_ref, mm_ref, x_ref, o_ref):
    sc = sc_ref[...]
    mm = mm_ref[...]
    v_lo = sc[:, 0:1]
    v_hi = sc[:, 1:2]
    v_mn = mm[:, 0:1]
    v_mx = mm[:, 1:2]
    same = v_hi == v_lo
    top0 = jnp.where(same, v_mx, v_hi)
    bot0 = jnp.where(same, v_mn, v_lo)
    all_black = jnp.any(top0 == 0.0)
    all_const = jnp.any(top0 == bot0)
    top = jnp.where(all_black, jnp.float32(1.0), top0)
    bot = jnp.where(jnp.logical_and(jnp.logical_not(all_black), all_const),
                    jnp.float32(0.0), bot0)
    r = pl.program_id(0)
    rows = lax.broadcasted_iota(jnp.int32, (N_ROWS, 1), 0)
    sel = rows == r
    top_r = jnp.sum(jnp.where(sel, top, 0.0))
    bot_r = jnp.sum(jnp.where(sel, bot, 0.0))
    xb = x_ref[...]
    o_ref[...] = jnp.clip((xb - bot_r) / (top_r - bot_r), 0.0, 1.0)


@jax.jit
def kernel(x):
    key, hpk, mm = pl.pallas_call(
        _prep_body,
        grid=(N_ROWS,),
        in_specs=[pl.BlockSpec((1, 512, 512), lambda i: (i, 0, 0))],
        out_specs=[
            pl.BlockSpec((1, 512, 512), lambda i: (i, 0, 0)),
            pl.BlockSpec((1, 256, 512), lambda i: (i, 0, 0)),
            pl.BlockSpec((1, 1, 128), lambda i: (i, 0, 0)),
        ],
        out_shape=(
            jax.ShapeDtypeStruct((N_ROWS, 512, 512), jnp.int32),
            jax.ShapeDtypeStruct((N_ROWS, 256, 512), jnp.int32),
            jax.ShapeDtypeStruct((N_ROWS, 1, 128), jnp.float32),
        ),
    )(x)
    sc = _select(hpk.reshape(-1), key.reshape(-1)).reshape(N_ROWS, 16)
    sc_pad = jnp.concatenate(
        [sc, jnp.zeros((N_ROWS, 112), jnp.float32)], axis=1)
    return pl.pallas_call(
        _norm_body,
        grid=(N_ROWS,),
        in_specs=[
            pl.BlockSpec((N_ROWS, 128), lambda i: (0, 0)),
            pl.BlockSpec((N_ROWS, 128), lambda i: (0, 0)),
            pl.BlockSpec((1, 512, 512), lambda i: (i, 0, 0)),
        ],
        out_specs=pl.BlockSpec((1, 512, 512), lambda i: (i, 0, 0)),
        out_shape=jax.ShapeDtypeStruct(x.shape, jnp.float32),
    )(sc_pad, mm.reshape(N_ROWS, 128), x)
